# baseline (speedup 1.0000x reference)
import functools

import jax
import jax.numpy as jnp
from jax import lax
from jax.experimental import pallas as pl
from jax.experimental.pallas import tpu as pltpu
from jax.experimental.pallas import tpu_sc as plsc

NUM_TOKENS = 1000000
D = 64
B = 4096
L = 200

NC = 2
NS = 16
NW = NC * NS
B_PER_W = B // NW
CB = 4
N_CHUNKS = B_PER_W // CB
IDX_PER_CHUNK = CB * L
G = 80
NG = IDX_PER_CHUNK // G

_mesh = plsc.VectorSubcoreMesh(core_axis_name="c", subcore_axis_name="s")


@functools.partial(
    pl.kernel,
    mesh=_mesh,
    out_type=jax.ShapeDtypeStruct((B, D), jnp.float32),
    compiler_params=pltpu.CompilerParams(use_tc_tiling_on_sc=False),
    scratch_types=[
        pltpu.VMEM((IDX_PER_CHUNK,), jnp.int32),
        pltpu.VMEM((IDX_PER_CHUNK, D), jnp.float32),
        pltpu.VMEM((B_PER_W, D), jnp.float32),
        pltpu.SemaphoreType.DMA,
    ],
)
def _emb_kernel(idx_hbm, table_hbm, out_hbm, idx_v, rows_v, out_v, sem):
    wid = lax.axis_index("s") * NC + lax.axis_index("c")
    idx_base = wid * (B_PER_W * L)

    def chunk(g, _):
        pltpu.sync_copy(
            idx_hbm.at[pl.ds(idx_base + g * IDX_PER_CHUNK, IDX_PER_CHUNK)], idx_v
        )
        copies = [
            pltpu.async_copy(
                table_hbm.at[idx_v.at[pl.ds(j * G, G)]],
                rows_v.at[pl.ds(j * G, G)],
                sem,
            )
            for j in range(NG)
        ]
        for c in copies:
            c.wait()
        for s in range(CB):
            def red(l, accs, s=s):
                r = s * L + l
                return tuple(
                    accs[j] + rows_v[r, pl.ds(j * 16, 16)] for j in range(D // 16)
                )
            accs = lax.fori_loop(
                0, L, red,
                tuple(jnp.zeros((16,), jnp.float32) for _ in range(D // 16)),
            )
            for j in range(D // 16):
                out_v[g * CB + s, pl.ds(j * 16, 16)] = accs[j]
        return _

    lax.fori_loop(0, N_CHUNKS, chunk, None)
    pltpu.sync_copy(out_v, out_hbm.at[pl.ds(wid * B_PER_W, B_PER_W)])


def kernel(inputs, table):
    idx_flat = inputs.astype(jnp.int32).reshape(-1)
    return _emb_kernel(idx_flat, table)


# speedup vs baseline: 1.1466x; 1.1466x over previous
import functools

import jax
import jax.numpy as jnp
from jax import lax
from jax.experimental import pallas as pl
from jax.experimental.pallas import tpu as pltpu
from jax.experimental.pallas import tpu_sc as plsc

NUM_TOKENS = 1000000
D = 64
B = 4096
L = 200

NC = 2
NS = 16
NW = NC * NS
B_PER_W = B // NW
CB = 4
N_CHUNKS = B_PER_W // CB
IDX_PER_CHUNK = CB * L
G = 80
NG = IDX_PER_CHUNK // G

_mesh = plsc.VectorSubcoreMesh(core_axis_name="c", subcore_axis_name="s")


@functools.partial(
    pl.kernel,
    mesh=_mesh,
    out_type=jax.ShapeDtypeStruct((B, D), jnp.float32),
    compiler_params=pltpu.CompilerParams(use_tc_tiling_on_sc=False),
    scratch_types=[
        pltpu.VMEM((IDX_PER_CHUNK,), jnp.int32),
        pltpu.VMEM((IDX_PER_CHUNK,), jnp.int32),
        pltpu.VMEM((IDX_PER_CHUNK, D), jnp.float32),
        pltpu.VMEM((IDX_PER_CHUNK, D), jnp.float32),
        pltpu.VMEM((B_PER_W, D), jnp.float32),
        pltpu.SemaphoreType.DMA,
        pltpu.SemaphoreType.DMA,
    ],
)
def _emb_kernel(idx_hbm, table_hbm, out_hbm, idx0_v, idx1_v, rows0_v, rows1_v,
                out_v, sem0, sem1):
    wid = lax.axis_index("s") * NC + lax.axis_index("c")
    idx_base = wid * (B_PER_W * L)

    def stage(g, idx_v, rows_v, sem):
        pltpu.sync_copy(
            idx_hbm.at[pl.ds(idx_base + g * IDX_PER_CHUNK, IDX_PER_CHUNK)], idx_v
        )
        for j in range(NG):
            pltpu.async_copy(
                table_hbm.at[idx_v.at[pl.ds(j * G, G)]],
                rows_v.at[pl.ds(j * G, G)],
                sem,
            )

    def drain(rows_v, sem):
        for j in range(NG):
            pltpu.make_async_copy(
                table_hbm.at[idx0_v.at[pl.ds(j * G, G)]],
                rows_v.at[pl.ds(j * G, G)],
                sem,
            ).wait()

    def reduce_chunk(g, rows_v):
        for s in range(CB):
            def red(t, accs, s=s):
                base = s * L + t * 8
                a0, a1, a2, a3 = accs
                for u in range(8):
                    r = base + u
                    a0 = a0 + rows_v[r, pl.ds(0, 16)]
                    a1 = a1 + rows_v[r, pl.ds(16, 16)]
                    a2 = a2 + rows_v[r, pl.ds(32, 16)]
                    a3 = a3 + rows_v[r, pl.ds(48, 16)]
                return (a0, a1, a2, a3)
            accs = lax.fori_loop(
                0, L // 8, red,
                tuple(jnp.zeros((16,), jnp.float32) for _ in range(D // 16)),
            )
            for j in range(D // 16):
                out_v[g * CB + s, pl.ds(j * 16, 16)] = accs[j]

    stage(0, idx0_v, rows0_v, sem0)

    def pair(h, _):
        g0 = h * 2
        stage(g0 + 1, idx1_v, rows1_v, sem1)
        drain(rows0_v, sem0)
        reduce_chunk(g0, rows0_v)

        @pl.when(h < N_CHUNKS // 2 - 1)
        def _prefetch():
            stage(g0 + 2, idx0_v, rows0_v, sem0)

        drain(rows1_v, sem1)
        reduce_chunk(g0 + 1, rows1_v)
        return _

    lax.fori_loop(0, N_CHUNKS // 2, pair, None)
    pltpu.sync_copy(out_v, out_hbm.at[pl.ds(wid * B_PER_W, B_PER_W)])


def kernel(inputs, table):
    idx_flat = inputs.astype(jnp.int32).reshape(-1)
    return _emb_kernel(idx_flat, table)
